# baseline (speedup 1.0000x reference)
import functools

import jax
import jax.numpy as jnp
from jax import lax
from jax.experimental import pallas as pl
from jax.experimental.pallas import tpu as pltpu
from jax.experimental.pallas import tpu_sc as plsc

B, S, D = 4, 4096, 4096
H1, H2 = D // 2, D // 4
BM = 1024
BK = 256


def _chunked_dot(a, w):
    s = None
    for i in range(a.shape[1] // BK):
        p = jnp.dot(a[:, i * BK:(i + 1) * BK], w[i * BK:(i + 1) * BK],
                    preferred_element_type=jnp.float32)
        s = p if s is None else s + p
    return s


def _score_body(x_ref, w1_ref, b1_ref, w2_ref, b2_ref, w3_ref, b3_ref,
                out_ref, h1_acc):
    k = pl.program_id(1)
    nk = pl.num_programs(1)

    @pl.when(k == 0)
    def _():
        h1_acc[...] = jnp.zeros_like(h1_acc)

    h1_acc[...] += jnp.dot(x_ref[...], w1_ref[...],
                           preferred_element_type=jnp.float32)

    @pl.when(k == nk - 1)
    def _():
        h1 = jnp.maximum(h1_acc[...] + b1_ref[...], 0.0)
        h2 = jnp.maximum(_chunked_dot(h1, w2_ref[...]) + b2_ref[...], 0.0)
        h2b = h2.astype(jnp.bfloat16).astype(jnp.float32)
        s = _chunked_dot(h2b, w3_ref[...]) + b3_ref[...]
        out_ref[...] = jax.nn.sigmoid(s).reshape(out_ref.shape)


def _scores(flat, W1, b1, W2, b2, W3, b3):
    m_tiles = (B * S) // BM
    k_tiles = D // BK
    out = pl.pallas_call(

        _score_body,
        grid=(m_tiles, k_tiles),
        in_specs=[
            pl.BlockSpec((BM, BK), lambda m, k: (m, k)),
            pl.BlockSpec((BK, H1), lambda m, k: (k, 0)),
            pl.BlockSpec((1, H1), lambda m, k: (0, 0)),
            pl.BlockSpec((H1, H2), lambda m, k: (0, 0)),
            pl.BlockSpec((1, H2), lambda m, k: (0, 0)),
            pl.BlockSpec((H2, 1), lambda m, k: (0, 0)),
            pl.BlockSpec((1, 1), lambda m, k: (0, 0)),
        ],
        out_specs=pl.BlockSpec((1, 1, BM), lambda m, k: (m, 0, 0)),
        out_shape=jax.ShapeDtypeStruct((m_tiles, 1, BM), jnp.float32),
        scratch_shapes=[pltpu.VMEM((BM, H1), jnp.float32)],
    )(flat, W1, b1.reshape(1, H1), W2, b2.reshape(1, H2), W3,
      b3.reshape(1, 1))
    return out.reshape(B, S)


_RT = 512


def _rank_body(s_ref, idx_ref, flat_ref, ranks_ref):
    b = pl.program_id(0)
    s_row = s_ref[0]
    jdx = lax.broadcasted_iota(jnp.int32, (_RT, S), 1)
    for t in range(S // _RT):
        si = s_ref[0, 0:1, t * _RT:(t + 1) * _RT].reshape(_RT, 1)
        idx_i = t * _RT + lax.broadcasted_iota(jnp.int32, (_RT, S), 0)
        gt = (s_row > si).astype(jnp.int32)
        eq_lt = ((s_row == si) & (jdx < idx_i)).astype(jnp.int32)
        ranks_ref[0, t * _RT:(t + 1) * _RT] = jnp.sum(gt + eq_lt, axis=1)
    ranks_row = ranks_ref[...]
    for t in range(S // _RT):
        r_i = t * _RT + lax.broadcasted_iota(jnp.int32, (_RT, S), 0)
        hit = ranks_row == r_i
        inv = jnp.sum(jnp.where(hit, jdx, 0), axis=1)
        idx_ref[0, 0, t * _RT:(t + 1) * _RT] = inv
        flat_ref[0, 0, t * _RT:(t + 1) * _RT] = inv + b * S


def _rank_invert(final_scores):
    idx3, flat3 = pl.pallas_call(
        _rank_body,
        grid=(B,),
        in_specs=[pl.BlockSpec((1, 1, S), lambda b: (b, 0, 0))],
        out_specs=[pl.BlockSpec((1, 1, S), lambda b: (b, 0, 0)),
                   pl.BlockSpec((1, 1, S), lambda b: (b, 0, 0))],
        out_shape=[jax.ShapeDtypeStruct((B, 1, S), jnp.int32),
                   jax.ShapeDtypeStruct((B, 1, S), jnp.int32)],
        scratch_shapes=[pltpu.VMEM((1, S), jnp.int32)],
    )(final_scores.reshape(B, 1, S))
    return idx3.reshape(B, S), flat3.reshape(B, S)


_NW = 32
_CHUNK = 16


def _gather_sc(table, flat_idx):
    total = B * S
    per_w = total // _NW
    mesh = plsc.VectorSubcoreMesh(core_axis_name="c", subcore_axis_name="s")

    @functools.partial(
        pl.kernel,
        mesh=mesh,
        out_type=jax.ShapeDtypeStruct((total, D), jnp.float32),
        scratch_types=[
            pltpu.VMEM((_CHUNK,), jnp.int32),
            pltpu.VMEM((_CHUNK, D), jnp.float32),
            pltpu.SemaphoreType.DMA,
        ],
    )
    def k(table_hbm, idx_hbm, out_hbm, idx_v, rows_v, sem):
        wid = lax.axis_index("s") * 2 + lax.axis_index("c")
        base = wid * per_w

        def body(c, carry):
            off = base + c * _CHUNK
            pltpu.sync_copy(idx_hbm.at[pl.ds(off, _CHUNK)], idx_v)
            pltpu.async_copy(table_hbm.at[idx_v], rows_v, sem).wait()
            pltpu.sync_copy(rows_v, out_hbm.at[pl.ds(off, _CHUNK)])
            return carry

        lax.fori_loop(0, per_w // _CHUNK, body, 0)

    return k(table, flat_idx)


def kernel(embeddings, W1, b1, W2, b2, W3, b3, sparsity_ratio):
    flat = embeddings.reshape(B * S, D)
    learned = _scores(flat, W1, b1, W2, b2, W3, b3)
    final_scores = learned * jnp.asarray(sparsity_ratio, jnp.float32)
    top_indices, flat_idx = _rank_invert(final_scores)
    selected = _gather_sc(flat, flat_idx.reshape(-1))
    return selected.reshape(B, S, D), top_indices, final_scores


# speedup vs baseline: 1.0299x; 1.0299x over previous
import functools

import jax
import jax.numpy as jnp
from jax import lax
from jax.experimental import pallas as pl
from jax.experimental.pallas import tpu as pltpu
from jax.experimental.pallas import tpu_sc as plsc

B, S, D = 4, 4096, 4096
H1, H2 = D // 2, D // 4
BM = 1024
BK = 256


def _chunked_dot(a, w):
    s = None
    for i in range(a.shape[1] // BK):
        p = jnp.dot(a[:, i * BK:(i + 1) * BK], w[i * BK:(i + 1) * BK],
                    preferred_element_type=jnp.float32)
        s = p if s is None else s + p
    return s


def _score_body(x_ref, w1_ref, b1_ref, w2_ref, b2_ref, w3_ref, b3_ref,
                out_ref, h1_acc):
    k = pl.program_id(1)
    nk = pl.num_programs(1)

    @pl.when(k == 0)
    def _():
        h1_acc[...] = jnp.zeros_like(h1_acc)

    h1_acc[...] += jnp.dot(x_ref[...], w1_ref[...],
                           preferred_element_type=jnp.float32)

    @pl.when(k == nk - 1)
    def _():
        h1 = jnp.maximum(h1_acc[...] + b1_ref[...], 0.0)
        h2 = jnp.maximum(_chunked_dot(h1, w2_ref[...]) + b2_ref[...], 0.0)
        h2b = h2.astype(jnp.bfloat16).astype(jnp.float32)
        s = _chunked_dot(h2b, w3_ref[...]) + b3_ref[...]
        out_ref[...] = jax.nn.sigmoid(s).reshape(out_ref.shape)


def _scores(flat, W1, b1, W2, b2, W3, b3):
    m_tiles = (B * S) // BM
    k_tiles = D // BK
    out = pl.pallas_call(

        _score_body,
        grid=(m_tiles, k_tiles),
        in_specs=[
            pl.BlockSpec((BM, BK), lambda m, k: (m, k)),
            pl.BlockSpec((BK, H1), lambda m, k: (k, 0)),
            pl.BlockSpec((1, H1), lambda m, k: (0, 0)),
            pl.BlockSpec((H1, H2), lambda m, k: (0, 0)),
            pl.BlockSpec((1, H2), lambda m, k: (0, 0)),
            pl.BlockSpec((H2, 1), lambda m, k: (0, 0)),
            pl.BlockSpec((1, 1), lambda m, k: (0, 0)),
        ],
        out_specs=pl.BlockSpec((1, 1, BM), lambda m, k: (m, 0, 0)),
        out_shape=jax.ShapeDtypeStruct((m_tiles, 1, BM), jnp.float32),
        scratch_shapes=[pltpu.VMEM((BM, H1), jnp.float32)],
    )(flat, W1, b1.reshape(1, H1), W2, b2.reshape(1, H2), W3,
      b3.reshape(1, 1))
    return out.reshape(B, S)


_RT = 512


def _rank_body(s_ref, idx_ref, flat_ref, ranks_ref):
    b = pl.program_id(0)
    s_row = s_ref[0]
    jdx = lax.broadcasted_iota(jnp.int32, (_RT, S), 1)
    for t in range(S // _RT):
        si = s_ref[0, 0:1, t * _RT:(t + 1) * _RT].reshape(_RT, 1)
        idx_i = t * _RT + lax.broadcasted_iota(jnp.int32, (_RT, S), 0)
        gt = (s_row > si).astype(jnp.int32)
        eq_lt = ((s_row == si) & (jdx < idx_i)).astype(jnp.int32)
        ranks_ref[0, t * _RT:(t + 1) * _RT] = jnp.sum(gt + eq_lt, axis=1)
    ranks_row = ranks_ref[...]
    for t in range(S // _RT):
        r_i = t * _RT + lax.broadcasted_iota(jnp.int32, (_RT, S), 0)
        hit = ranks_row == r_i
        inv = jnp.sum(jnp.where(hit, jdx, 0), axis=1)
        idx_ref[0, 0, t * _RT:(t + 1) * _RT] = inv
        flat_ref[0, 0, t * _RT:(t + 1) * _RT] = inv + b * S


def _rank_invert(final_scores):
    idx3, flat3 = pl.pallas_call(
        _rank_body,
        grid=(B,),
        in_specs=[pl.BlockSpec((1, 1, S), lambda b: (b, 0, 0))],
        out_specs=[pl.BlockSpec((1, 1, S), lambda b: (b, 0, 0)),
                   pl.BlockSpec((1, 1, S), lambda b: (b, 0, 0))],
        out_shape=[jax.ShapeDtypeStruct((B, 1, S), jnp.int32),
                   jax.ShapeDtypeStruct((B, 1, S), jnp.int32)],
        scratch_shapes=[pltpu.VMEM((1, S), jnp.int32)],
    )(final_scores.reshape(B, 1, S))
    return idx3.reshape(B, S), flat3.reshape(B, S)


_NW = 32
_CHUNK = 8


def _gather_sc(table, flat_idx):
    total = B * S
    per_w = total // _NW
    mesh = plsc.VectorSubcoreMesh(core_axis_name="c", subcore_axis_name="s")

    nch = per_w // _CHUNK

    @functools.partial(
        pl.kernel,
        mesh=mesh,
        out_type=jax.ShapeDtypeStruct((total, D), jnp.float32),
        scratch_types=[
            pltpu.VMEM((_CHUNK,), jnp.int32),
            pltpu.VMEM((_CHUNK,), jnp.int32),
            pltpu.VMEM((_CHUNK, D), jnp.float32),
            pltpu.VMEM((_CHUNK, D), jnp.float32),
            pltpu.SemaphoreType.DMA,
            pltpu.SemaphoreType.DMA,
        ],
    )
    def k(table_hbm, idx_hbm, out_hbm, idx0, idx1, rows0, rows1, s0, s1):
        wid = lax.axis_index("s") * 2 + lax.axis_index("c")
        base = wid * per_w
        bufs = ((idx0, rows0, s0), (idx1, rows1, s1))

        def start(c, b):
            idx_v, rows_v, sem = bufs[b]
            pltpu.sync_copy(idx_hbm.at[pl.ds(base + c * _CHUNK, _CHUNK)],
                            idx_v)
            pltpu.async_copy(table_hbm.at[idx_v], rows_v, sem)

        def drain(c, b):
            idx_v, rows_v, sem = bufs[b]
            pltpu.make_async_copy(table_hbm.at[idx_v], rows_v, sem).wait()
            pltpu.sync_copy(rows_v,
                            out_hbm.at[pl.ds(base + c * _CHUNK, _CHUNK)])

        start(0, 0)

        def body(i, carry):
            c0 = i * 2
            start(c0 + 1, 1)
            drain(c0, 0)

            @pl.when(c0 + 2 < nch)
            def _():
                start(c0 + 2, 0)

            drain(c0 + 1, 1)
            return carry

        lax.fori_loop(0, nch // 2, body, 0)

    return k(table, flat_idx)


def kernel(embeddings, W1, b1, W2, b2, W3, b3, sparsity_ratio):
    flat = embeddings.reshape(B * S, D)
    learned = _scores(flat, W1, b1, W2, b2, W3, b3)
    final_scores = learned * jnp.asarray(sparsity_ratio, jnp.float32)
    top_indices, flat_idx = _rank_invert(final_scores)
    selected = _gather_sc(flat, flat_idx.reshape(-1))
    return selected.reshape(B, S, D), top_indices, final_scores


# speedup vs baseline: 1.0570x; 1.0263x over previous
import functools

import jax
import jax.numpy as jnp
from jax import lax
from jax.experimental import pallas as pl
from jax.experimental.pallas import tpu as pltpu
from jax.experimental.pallas import tpu_sc as plsc

B, S, D = 4, 4096, 4096
H1, H2 = D // 2, D // 4
BM = 1024
BK = 256


def _chunked_dot(a, w):
    s = None
    for i in range(a.shape[1] // BK):
        p = jnp.dot(a[:, i * BK:(i + 1) * BK], w[i * BK:(i + 1) * BK],
                    preferred_element_type=jnp.float32)
        s = p if s is None else s + p
    return s


def _score_body(x_ref, w1_ref, b1_ref, w2_ref, b2_ref, w3_ref, b3_ref,
                out_ref, h1_acc):
    k = pl.program_id(1)
    nk = pl.num_programs(1)

    @pl.when(k == 0)
    def _():
        h1_acc[...] = jnp.zeros_like(h1_acc)

    h1_acc[...] += jnp.dot(x_ref[...], w1_ref[...],
                           preferred_element_type=jnp.float32)

    @pl.when(k == nk - 1)
    def _():
        h1 = jnp.maximum(h1_acc[...] + b1_ref[...], 0.0)
        h2 = jnp.maximum(_chunked_dot(h1, w2_ref[...]) + b2_ref[...], 0.0)
        h2b = h2.astype(jnp.bfloat16).astype(jnp.float32)
        s = _chunked_dot(h2b, w3_ref[...]) + b3_ref[...]
        out_ref[...] = jax.nn.sigmoid(s).reshape(out_ref.shape)


def _scores(flat, W1, b1, W2, b2, W3, b3):
    m_tiles = (B * S) // BM
    k_tiles = D // BK
    out = pl.pallas_call(

        _score_body,
        grid=(m_tiles, k_tiles),
        in_specs=[
            pl.BlockSpec((BM, BK), lambda m, k: (m, k)),
            pl.BlockSpec((BK, H1), lambda m, k: (k, 0)),
            pl.BlockSpec((1, H1), lambda m, k: (0, 0)),
            pl.BlockSpec((H1, H2), lambda m, k: (0, 0)),
            pl.BlockSpec((1, H2), lambda m, k: (0, 0)),
            pl.BlockSpec((H2, 1), lambda m, k: (0, 0)),
            pl.BlockSpec((1, 1), lambda m, k: (0, 0)),
        ],
        out_specs=pl.BlockSpec((1, 1, BM), lambda m, k: (m, 0, 0)),
        out_shape=jax.ShapeDtypeStruct((m_tiles, 1, BM), jnp.float32),
        scratch_shapes=[pltpu.VMEM((BM, H1), jnp.float32)],
    )(flat, W1, b1.reshape(1, H1), W2, b2.reshape(1, H2), W3,
      b3.reshape(1, 1))
    return out.reshape(B, S)


_RT = 512


def _rank_body(s_ref, rank_ref):
    b = pl.program_id(0)
    s_row = s_ref[0]
    jdx = lax.broadcasted_iota(jnp.int32, (_RT, S), 1)
    for t in range(S // _RT):
        si = s_ref[0, 0:1, t * _RT:(t + 1) * _RT].reshape(_RT, 1)
        idx_i = t * _RT + lax.broadcasted_iota(jnp.int32, (_RT, S), 0)
        gt = (s_row > si).astype(jnp.int32)
        eq_lt = ((s_row == si) & (jdx < idx_i)).astype(jnp.int32)
        rank_ref[0, 0, t * _RT:(t + 1) * _RT] = \
            jnp.sum(gt + eq_lt, axis=1) + b * S


def _ranks(final_scores):
    r3 = pl.pallas_call(
        _rank_body,
        grid=(B,),
        in_specs=[pl.BlockSpec((1, 1, S), lambda b: (b, 0, 0))],
        out_specs=pl.BlockSpec((1, 1, S), lambda b: (b, 0, 0)),
        out_shape=jax.ShapeDtypeStruct((B, 1, S), jnp.int32),
    )(final_scores.reshape(B, 1, S))
    return r3.reshape(B * S)


_NW = 32
_CHUNK = 8


def _scatter_sc(table, flat_rank, positions):
    total = B * S
    per_w = total // _NW
    mesh = plsc.VectorSubcoreMesh(core_axis_name="c", subcore_axis_name="s")

    nch = per_w // _CHUNK

    @functools.partial(
        pl.kernel,
        mesh=mesh,
        out_type=[jax.ShapeDtypeStruct((total, D), jnp.float32),
                  jax.ShapeDtypeStruct((total,), jnp.int32)],
        scratch_types=[
            pltpu.VMEM((_CHUNK,), jnp.int32),
            pltpu.VMEM((_CHUNK,), jnp.int32),
            pltpu.VMEM((_CHUNK,), jnp.int32),
            pltpu.VMEM((_CHUNK,), jnp.int32),
            pltpu.VMEM((_CHUNK, D), jnp.float32),
            pltpu.VMEM((_CHUNK, D), jnp.float32),
            pltpu.SemaphoreType.DMA,
            pltpu.SemaphoreType.DMA,
            pltpu.SemaphoreType.DMA,
            pltpu.SemaphoreType.DMA,
            pltpu.SemaphoreType.DMA,
            pltpu.SemaphoreType.DMA,
        ],
    )
    def k(table_hbm, rank_hbm, pos_hbm, out_hbm, tidx_hbm,
          rk0, rk1, ps0, ps1, rows0, rows1, sr0, sr1, sw0, sw1, st0, st1):
        wid = lax.axis_index("s") * 2 + lax.axis_index("c")
        base = wid * per_w
        bufs = ((rk0, ps0, rows0, sr0, sw0, st0),
                (rk1, ps1, rows1, sr1, sw1, st1))

        def start(c, b):
            rk, ps, rows_v, sr, sw, st = bufs[b]

            @pl.when(c >= 2)
            def _():
                pltpu.make_async_copy(rows_v, out_hbm.at[rk], sw).wait()
                pltpu.make_async_copy(ps, tidx_hbm.at[rk], st).wait()

            off = base + c * _CHUNK
            pltpu.sync_copy(rank_hbm.at[pl.ds(off, _CHUNK)], rk)
            pltpu.sync_copy(pos_hbm.at[pl.ds(off, _CHUNK)], ps)
            pltpu.async_copy(table_hbm.at[pl.ds(off, _CHUNK)], rows_v, sr)

        def finish(c, b):
            rk, ps, rows_v, sr, sw, st = bufs[b]
            pltpu.make_async_copy(
                table_hbm.at[pl.ds(base + c * _CHUNK, _CHUNK)],
                rows_v, sr).wait()
            pltpu.async_copy(rows_v, out_hbm.at[rk], sw)
            pltpu.async_copy(ps, tidx_hbm.at[rk], st)

        start(0, 0)

        def body(i, carry):
            c0 = i * 2
            start(c0 + 1, 1)
            finish(c0, 0)

            @pl.when(c0 + 2 < nch)
            def _():
                start(c0 + 2, 0)

            finish(c0 + 1, 1)
            return carry

        lax.fori_loop(0, nch // 2, body, 0)

        for b in range(2):
            rk, ps, rows_v, sr, sw, st = bufs[b]
            pltpu.make_async_copy(rows_v, out_hbm.at[rk], sw).wait()
            pltpu.make_async_copy(ps, tidx_hbm.at[rk], st).wait()

    return k(table, flat_rank, positions)


def kernel(embeddings, W1, b1, W2, b2, W3, b3, sparsity_ratio):
    flat = embeddings.reshape(B * S, D)
    learned = _scores(flat, W1, b1, W2, b2, W3, b3)
    final_scores = learned * jnp.asarray(sparsity_ratio, jnp.float32)
    flat_rank = _ranks(final_scores)
    positions = jnp.tile(jnp.arange(S, dtype=jnp.int32), B)
    selected, tidx = _scatter_sc(flat, flat_rank, positions)
    return selected.reshape(B, S, D), tidx.reshape(B, S), final_scores
